# baseline (speedup 1.0000x reference)
import functools

import numpy as np

import jax
import jax.numpy as jnp
from jax.experimental import pallas as pl
from jax.experimental.pallas import tpu as pltpu

_NREP = 512
_NCONV = 384
_NBIG = 512


def _tagger_kernel(idx_ref, s_ref, t_ref, wchar_ref, w1wc_ref,
                   w2_ref, aux_ref, out_ref, *, O, Od):
    rep = jnp.dot(idx_ref[...], s_ref[...],
                  preferred_element_type=jnp.float32).astype(jnp.bfloat16)
    oh = jnp.where(rep == t_ref[...], jnp.bfloat16(1.0), jnp.bfloat16(0.0))

    conv = jnp.dot(oh[:, 0:256], wchar_ref[...],
                   preferred_element_type=jnp.float32).astype(jnp.bfloat16)

    g0 = conv[:, 0:128]
    g1 = conv[:, 128:256]
    g2 = conv[:, 256:384]
    gm01 = jnp.maximum(g0, g1)
    gm = jnp.maximum(gm01, g2)
    cf = jnp.maximum(jnp.maximum(gm[:, 0:O], gm[:, O:2 * O]),
                     gm01[:, 2 * O:3 * O])

    h = jnp.tanh(jnp.dot(jnp.concatenate([oh[:, 256:512], cf], axis=1),
                         w1wc_ref[...], preferred_element_type=jnp.float32)
                 + aux_ref[1:2, :])

    logits = jnp.dot(h.astype(jnp.bfloat16), w2_ref[...],
                     preferred_element_type=jnp.float32) + aux_ref[2:3, :]
    lse = jnp.log(jnp.sum(jnp.exp(logits), axis=-1, keepdims=True))
    out_ref[...] = (logits - lse)[:, :Od]


@functools.partial(jax.jit, static_argnames=("tile_b", "interpret"))
def _forward(words_idxs, chars_idxs, char_emb, word_emb, conv_w, conv_b,
             W1, b1, W2, b2, *, tile_b=512, interpret=False):
    B, Cs = chars_idxs.shape
    Wn = words_idxs.shape[1]
    char_emb = char_emb.astype(jnp.float32)
    word_emb = word_emb.astype(jnp.float32)
    Vc, L = char_emb.shape
    Vw, E = word_emb.shape
    Wc = conv_w.astype(jnp.float32)
    O = Wc.shape[0]
    W1f = W1.astype(jnp.float32)
    W2f = W2.astype(jnp.float32)
    Od, H = W2f.shape
    Dw = Wn * E
    hi = jax.lax.Precision.HIGHEST

    S_np = np.zeros((16, 512), np.float32)
    T_np = np.full((1, 512), -1.0, np.float32)
    for c in range(Cs):
        S_np[c, c * Vc:(c + 1) * Vc] = 1.0
        T_np[0, c * Vc:(c + 1) * Vc] = np.arange(Vc)
    for w in range(Wn):
        S_np[Cs + w, 256 + w * Vw:256 + (w + 1) * Vw] = 1.0
        T_np[0, 256 + w * Vw:256 + (w + 1) * Vw] = np.arange(Vw)
    S_c = jnp.asarray(S_np, jnp.bfloat16)
    T_c = jnp.asarray(T_np, jnp.bfloat16)

    Es = jnp.stack([
        jnp.pad(char_emb[:, :L - 1], ((0, 0), (1, 0))),
        char_emb,
        jnp.pad(char_emb[:, 1:], ((0, 0), (0, 1))),
    ], axis=1)
    W3 = jnp.transpose(Wc, (2, 1, 0))
    T4 = jnp.einsum("vkm,kco->cvmo", Es, W3, precision=hi)
    T4 = jnp.pad(T4.reshape(Cs * Vc, L, O), ((0, 0), (0, 1), (0, 0)))
    T4 = jnp.pad(T4.reshape(Cs * Vc, 3, 3 * O),
                 ((0, 0), (0, 0), (0, 128 - 3 * O)))
    Wchar = jnp.pad(T4.reshape(Cs * Vc, _NCONV),
                    ((0, 256 - Cs * Vc), (0, 0))).astype(jnp.bfloat16)

    w1w = jnp.einsum("ve,hwe->wvh", word_emb, W1f[:, :Dw].reshape(H, Wn, E),
                     precision=hi).reshape(Wn * Vw, H)
    W1cf = W1f[:, Dw:].T
    W1wc = jnp.pad(
        jnp.concatenate([jnp.pad(w1w, ((0, 256 - Wn * Vw), (0, 0))), W1cf],
                        axis=0),
        ((0, 0), (0, 128 - H))).astype(jnp.bfloat16)
    W2p = jnp.pad(W2f.T, ((0, 128 - H), (0, 128 - Od))).astype(jnp.bfloat16)
    b1_eff = b1.astype(jnp.float32) + jnp.dot(
        conv_b.astype(jnp.float32), W1cf, precision=hi)
    aux = jnp.stack([
        jnp.zeros((128,), jnp.float32),
        jnp.pad(b1_eff, (0, 128 - H)),
        jnp.concatenate([b2.astype(jnp.float32),
                         jnp.full((128 - Od,), -1e30, jnp.float32)]),
    ] + [jnp.zeros((128,), jnp.float32)] * 5)

    TB = min(tile_b, B)
    grid_b = pl.cdiv(B, TB)

    idx_all = jnp.pad(
        jnp.concatenate([chars_idxs.astype(jnp.int32),
                         words_idxs.astype(jnp.int32)], axis=1),
        ((0, 0), (0, 16 - Cs - Wn))).astype(jnp.bfloat16)

    out = pl.pallas_call(
        functools.partial(_tagger_kernel, O=O, Od=Od),
        out_shape=jax.ShapeDtypeStruct((B, Od), jnp.float32),
        grid_spec=pltpu.PrefetchScalarGridSpec(
            num_scalar_prefetch=0,
            grid=(grid_b,),
            in_specs=[
                pl.BlockSpec((TB, 16), lambda b: (b, 0)),
                pl.BlockSpec((16, 512), lambda b: (0, 0)),
                pl.BlockSpec((1, 512), lambda b: (0, 0)),
                pl.BlockSpec((256, _NCONV), lambda b: (0, 0)),
                pl.BlockSpec((256 + O, 128), lambda b: (0, 0)),
                pl.BlockSpec((128, 128), lambda b: (0, 0)),
                pl.BlockSpec((8, 128), lambda b: (0, 0)),
            ],
            out_specs=pl.BlockSpec((TB, Od), lambda b: (b, 0)),
        ),
        compiler_params=pltpu.CompilerParams(
            dimension_semantics=("parallel",),
            allow_input_fusion=[True] * 7,
            flags={"XLA_TPU_STORE_TO_LOAD_FORWARDING_WINDOW": 12288}),
        interpret=interpret,
    )(idx_all, S_c, T_c, Wchar, W1wc, W2p, aux)
    return out


def kernel(words_idxs, chars_idxs, char_emb, word_emb, conv_w, conv_b,
           W1, b1, W2, b2):
    return _forward(words_idxs, chars_idxs, char_emb, word_emb,
                    conv_w, conv_b, W1, b1, W2, b2, tile_b=8192)


# speedup vs baseline: 1.0072x; 1.0072x over previous
import functools

import numpy as np

import jax
import jax.numpy as jnp
from jax.experimental import pallas as pl
from jax.experimental.pallas import tpu as pltpu

_NCONV = 384


def _tagger_kernel(idx_ref, s_ref, t_ref, wchar_ref, w1wc_ref,
                   w2_ref, aux_ref, out_ref, *, O, Od):
    rep = jnp.dot(idx_ref[...], s_ref[...],
                  preferred_element_type=jnp.float32).astype(jnp.bfloat16)
    oh = jnp.where(rep == t_ref[...], jnp.bfloat16(1.0), jnp.bfloat16(0.0))

    conv = jnp.dot(oh[:, 0:256], wchar_ref[...],
                   preferred_element_type=jnp.float32).astype(jnp.bfloat16)

    g0 = conv[:, 0:128]
    g1 = conv[:, 128:256]
    g2 = conv[:, 256:384]
    gm01 = jnp.maximum(g0, g1)
    gm = jnp.maximum(gm01, g2)
    cf = jnp.maximum(jnp.maximum(gm[:, 0:O], gm[:, O:2 * O]),
                     gm01[:, 2 * O:3 * O])

    h = jnp.tanh(jnp.dot(jnp.concatenate([oh[:, 256:512], cf], axis=1),
                         w1wc_ref[...], preferred_element_type=jnp.float32)
                 + aux_ref[1:2, :])

    logits = jnp.dot(h.astype(jnp.bfloat16), w2_ref[...],
                     preferred_element_type=jnp.float32) + aux_ref[2:3, :]
    lse = jnp.log(jnp.sum(jnp.exp(logits), axis=-1, keepdims=True))
    out_ref[...] = (logits - lse)[:, :Od]


@functools.partial(jax.jit, static_argnames=("tile_b", "interpret"))
def _forward(words_idxs, chars_idxs, char_emb, word_emb, conv_w, conv_b,
             W1, b1, W2, b2, *, tile_b=512, interpret=False):
    B, Cs = chars_idxs.shape
    Wn = words_idxs.shape[1]
    char_emb = char_emb.astype(jnp.float32)
    word_emb = word_emb.astype(jnp.float32)
    Vc, L = char_emb.shape
    Vw, E = word_emb.shape
    Wc = conv_w.astype(jnp.float32)
    O = Wc.shape[0]
    W1f = W1.astype(jnp.float32)
    W2f = W2.astype(jnp.float32)
    Od, H = W2f.shape
    Dw = Wn * E
    hi = jax.lax.Precision.HIGHEST

    S_np = np.zeros((16, 512), np.float32)
    T_np = np.full((1, 512), -1.0, np.float32)
    for c in range(Cs):
        S_np[c, c * Vc:(c + 1) * Vc] = 1.0
        T_np[0, c * Vc:(c + 1) * Vc] = np.arange(Vc)
    for w in range(Wn):
        S_np[Cs + w, 256 + w * Vw:256 + (w + 1) * Vw] = 1.0
        T_np[0, 256 + w * Vw:256 + (w + 1) * Vw] = np.arange(Vw)
    S_c = jnp.asarray(S_np, jnp.bfloat16)
    T_c = jnp.asarray(T_np, jnp.bfloat16)

    Es = jnp.stack([
        jnp.pad(char_emb[:, :L - 1], ((0, 0), (1, 0))),
        char_emb,
        jnp.pad(char_emb[:, 1:], ((0, 0), (0, 1))),
    ], axis=1)
    W3 = jnp.transpose(Wc, (2, 1, 0))
    T4 = jnp.einsum("vkm,kco->cvmo", Es, W3, precision=hi)
    T4 = jnp.pad(T4.reshape(Cs * Vc, L, O), ((0, 0), (0, 1), (0, 0)))
    T4 = jnp.pad(T4.reshape(Cs * Vc, 3, 3 * O),
                 ((0, 0), (0, 0), (0, 128 - 3 * O)))
    Wchar = jnp.pad(T4.reshape(Cs * Vc, _NCONV),
                    ((0, 256 - Cs * Vc), (0, 0))).astype(jnp.bfloat16)

    w1w = jnp.einsum("ve,hwe->wvh", word_emb, W1f[:, :Dw].reshape(H, Wn, E),
                     precision=hi).reshape(Wn * Vw, H)
    W1cf = W1f[:, Dw:].T
    W1wc = jnp.pad(
        jnp.concatenate([jnp.pad(w1w, ((0, 256 - Wn * Vw), (0, 0))), W1cf],
                        axis=0),
        ((0, 0), (0, 128 - H))).astype(jnp.bfloat16)
    W2p = jnp.pad(W2f.T, ((0, 128 - H), (0, 128 - Od))).astype(jnp.bfloat16)
    b1_eff = b1.astype(jnp.float32) + jnp.dot(
        conv_b.astype(jnp.float32), W1cf, precision=hi)
    aux = jnp.stack([
        jnp.zeros((128,), jnp.float32),
        jnp.pad(b1_eff, (0, 128 - H)),
        jnp.concatenate([b2.astype(jnp.float32),
                         jnp.full((128 - Od,), -1e30, jnp.float32)]),
    ] + [jnp.zeros((128,), jnp.float32)] * 5)

    TB = min(tile_b, B)
    grid_b = pl.cdiv(B, TB)

    idx_all = jnp.pad(
        jnp.concatenate([chars_idxs.astype(jnp.int32),
                         words_idxs.astype(jnp.int32)], axis=1),
        ((0, 0), (0, 16 - Cs - Wn))).astype(jnp.bfloat16)

    out = pl.pallas_call(
        functools.partial(_tagger_kernel, O=O, Od=Od),
        out_shape=jax.ShapeDtypeStruct((B, Od), jnp.float32),
        grid_spec=pltpu.PrefetchScalarGridSpec(
            num_scalar_prefetch=0,
            grid=(grid_b,),
            in_specs=[
                pl.BlockSpec((TB, 16), lambda b: (b, 0)),
                pl.BlockSpec((16, 512), lambda b: (0, 0)),
                pl.BlockSpec((1, 512), lambda b: (0, 0)),
                pl.BlockSpec((256, _NCONV), lambda b: (0, 0)),
                pl.BlockSpec((256 + O, 128), lambda b: (0, 0)),
                pl.BlockSpec((128, 128), lambda b: (0, 0)),
                pl.BlockSpec((8, 128), lambda b: (0, 0)),
            ],
            out_specs=pl.BlockSpec((TB, Od), lambda b: (b, 0)),
        ),
        compiler_params=pltpu.CompilerParams(
            dimension_semantics=("parallel",),
            allow_input_fusion=[True] + [False] * 6,
            flags={"XLA_TPU_STORE_TO_LOAD_FORWARDING_WINDOW": 12288}),
        interpret=interpret,
    )(idx_all, S_c, T_c, Wchar, W1wc, W2p, aux)
    return out


def kernel(words_idxs, chars_idxs, char_emb, word_emb, conv_w, conv_b,
           W1, b1, W2, b2):
    return _forward(words_idxs, chars_idxs, char_emb, word_emb,
                    conv_w, conv_b, W1, b1, W2, b2, tile_b=8192)
